# baseline (speedup 1.0000x reference)
import jax
import jax.numpy as jnp
from jax import lax
from jax.experimental import pallas as pl
from jax.experimental.pallas import tpu as pltpu
from jax.experimental.pallas import tpu_sc as plsc

N, K, H, W = 1, 16, 64, 64
L = 16
Q = W // L


def _decode_kernel(hm_hbm, cen_hbm, scl_hbm, out_hbm, mv_hbm,
                   hm_v, cen_v, scl_v, res_v, big_v, stage_v, mv_v, shared,
                   sem0, sem1, sem2):
    s = lax.axis_index("s")

    cp_hm1 = pltpu.async_copy(
        hm_hbm.at[0, s, pl.ds(0, H // 2)], hm_v.at[pl.ds(0, H // 2)], sem0)
    cp_hm2 = pltpu.async_copy(
        hm_hbm.at[0, s, pl.ds(H // 2, H // 2)], hm_v.at[pl.ds(H // 2, H // 2)],
        sem1)
    cp_c = pltpu.async_copy(cen_hbm, cen_v, sem2)
    cp_s = pltpu.async_copy(scl_hbm, scl_v, sem2)

    lanes = lax.broadcasted_iota(jnp.int32, (L,), 0)
    neg = jnp.full((L,), -jnp.inf, jnp.float32)
    zeroi = jnp.zeros((L,), jnp.int32)

    def body(r, carry):
        new = []
        for q in range(Q):
            rmax, rrow = carry[2 * q], carry[2 * q + 1]
            v = hm_v[r, pl.ds(q * L, L)]
            take = v > rmax
            new.append(jnp.maximum(v, rmax))
            new.append(jnp.where(take, r, rrow))
        return tuple(new)

    cp_hm1.wait()
    acc = lax.fori_loop(0, H // 2, body, (neg, zeroi) * Q)
    cp_hm2.wait()
    acc = lax.fori_loop(H // 2, H, body, acc)

    vals = [acc[2 * q] for q in range(Q)]
    idxs = [acc[2 * q + 1] * W + (q * L) + lanes for q in range(Q)]
    n = Q
    while n > 1:
        n //= 2
        for q in range(n):
            v1, i1 = vals[q], idxs[q]
            v2, i2 = vals[q + n], idxs[q + n]
            take2 = (v2 > v1) | ((v2 == v1) & (i2 < i1))
            vals[q] = jnp.where(take2, v2, v1)
            idxs[q] = jnp.where(take2, i2, i1)
    rmax, rflat = vals[0], idxs[0]

    m = jnp.max(rmax)
    cand = jnp.where(rmax == m, rflat, H * W)
    idx = jnp.min(cand)

    valid = m > 0.0
    px = jnp.where(valid, idx & (W - 1), -1)
    py = jnp.where(valid, idx >> 6, -1)
    pxf = px.astype(jnp.float32)
    pyf = py.astype(jnp.float32)

    inb = (px > 1) & (px < W - 1) & (py > 1) & (py < H - 1)
    pxc = jnp.clip(px, 1, W - 2)
    pyc = jnp.clip(py, 1, H - 2)

    rowv = pyc + jnp.where(lanes == 2, 1, jnp.where(lanes == 3, -1, 0))
    colv = pxc + jnp.where(lanes == 0, 1, jnp.where(lanes == 1, -1, 0))
    v4 = plsc.load_gather(hm_v, [rowv, colv])
    zero = jnp.zeros((L,), jnp.float32)
    dx = jnp.sum(jnp.where(lanes == 0, v4, jnp.where(lanes == 1, -v4, zero)))
    dy = jnp.sum(jnp.where(lanes == 2, v4, jnp.where(lanes == 3, -v4, zero)))

    rx = pxf + jnp.where(inb, jnp.sign(dx) * 0.25, 0.0)
    ry = pyf + jnp.where(inb, jnp.sign(dy) * 0.25, 0.0)

    cp_c.wait()
    cp_s.wait()
    zer16 = jnp.zeros((L,), jnp.int32)
    pair = jnp.minimum(lanes, 1)
    cv = plsc.load_gather(cen_v, [zer16, pair])
    sv = plsc.load_gather(scl_v, [zer16, pair])
    cx = cv[0]
    cy = cv[1]
    scx = sv[0] * 200.0
    scy = sv[1] * 200.0
    tx = rx * (scx * (1.0 / W)) + cx - scx * 0.5
    ty = ry * (scy * (1.0 / H)) + cy - scy * 0.5

    res_v[...] = jnp.where(
        lanes == 0, tx,
        jnp.where(lanes == 1, ty, jnp.where(lanes == 2, m, zero)))
    pltpu.sync_copy(res_v, shared.at[pl.ds(s * L, L)])
    plsc.subcore_barrier()

    @pl.when(s == 0)
    def _():
        pltpu.sync_copy(shared, big_v)
        half = (lanes >> 1) * L + (lanes & 1)
        ab = plsc.load_gather(big_v, [half])
        cd = plsc.load_gather(big_v, [half + 8 * L])
        row = lanes >> 1
        col = lanes & 1
        plsc.store_scatter(stage_v, [row, col], ab)
        plsc.store_scatter(stage_v, [row + 8, col], cd)
        pltpu.sync_copy(stage_v, out_hbm.at[0])

    @pl.when(s == 1)
    def _():
        pltpu.sync_copy(shared, big_v)
        mv = plsc.load_gather(big_v, [lanes * L + 2])
        plsc.store_scatter(mv_v, [lanes, jnp.zeros((L,), jnp.int32)], mv)
        pltpu.sync_copy(mv_v, mv_hbm.at[0])


@jax.jit
def kernel(heatmaps, center, scale):
    mesh = plsc.VectorSubcoreMesh(
        core_axis_name="c", subcore_axis_name="s", num_cores=1)
    run = pl.kernel(
        _decode_kernel,
        out_type=(jax.ShapeDtypeStruct((N, K, 2), jnp.float32),
                  jax.ShapeDtypeStruct((N, K, 1), jnp.float32)),
        mesh=mesh,
        scratch_types=[
            pltpu.VMEM((H, W), jnp.float32),
            pltpu.VMEM((1, 2), jnp.float32),
            pltpu.VMEM((1, 2), jnp.float32),
            pltpu.VMEM((L,), jnp.float32),
            pltpu.VMEM((K * L,), jnp.float32),
            pltpu.VMEM((K, 2), jnp.float32),
            pltpu.VMEM((K, 1), jnp.float32),
            pltpu.VMEM_SHARED((K * L,), jnp.float32),
            pltpu.SemaphoreType.DMA,
            pltpu.SemaphoreType.DMA,
            pltpu.SemaphoreType.DMA,
        ],
        compiler_params=pltpu.CompilerParams(needs_layout_passes=False),
    )
    return run(heatmaps, center, scale)


# speedup vs baseline: 1.0078x; 1.0078x over previous
import jax
import jax.numpy as jnp
from jax import lax
from jax.experimental import pallas as pl
from jax.experimental.pallas import tpu as pltpu
from jax.experimental.pallas import tpu_sc as plsc

N, K, H, W = 1, 16, 64, 64
L = 16
Q = W // L


def _decode_kernel(hm_hbm, cen_hbm, scl_hbm, out_hbm, mv_hbm,
                   hm_v, cen_v, scl_v, res_v, big_v, stage_v, shared,
                   sem0, sem1, sem2):
    s = lax.axis_index("s")

    cp_hm = pltpu.async_copy(hm_hbm.at[0, s], hm_v, sem0)
    cp_c = pltpu.async_copy(cen_hbm, cen_v, sem1)
    cp_s = pltpu.async_copy(scl_hbm, scl_v, sem2)
    cp_hm.wait()

    lanes = lax.broadcasted_iota(jnp.int32, (L,), 0)
    neg = jnp.full((L,), -jnp.inf, jnp.float32)
    zeroi = jnp.zeros((L,), jnp.int32)

    R = 2
    NA = R * Q

    def body(i, carry):
        new = []
        for p in range(R):
            r = i * R + p
            for q in range(Q):
                k = p * Q + q
                rmax, rrow = carry[2 * k], carry[2 * k + 1]
                v = hm_v[r, pl.ds(q * L, L)]
                take = v > rmax
                new.append(jnp.maximum(v, rmax))
                new.append(jnp.where(take, r, rrow))
        return tuple(new)

    acc = lax.fori_loop(0, H // R, body, (neg, zeroi) * NA)

    vals = [acc[2 * k] for k in range(NA)]
    idxs = [acc[2 * k + 1] * W + ((k % Q) * L) + lanes for k in range(NA)]
    n = NA
    while n > 1:
        n //= 2
        for k in range(n):
            v1, i1 = vals[k], idxs[k]
            v2, i2 = vals[k + n], idxs[k + n]
            take2 = (v2 > v1) | ((v2 == v1) & (i2 < i1))
            vals[k] = jnp.where(take2, v2, v1)
            idxs[k] = jnp.where(take2, i2, i1)
    rmax, rflat = vals[0], idxs[0]

    m = jnp.max(rmax)
    cand = jnp.where(rmax == m, rflat, H * W)
    idx = jnp.min(cand)

    valid = m > 0.0
    px = jnp.where(valid, idx & (W - 1), -1)
    py = jnp.where(valid, idx >> 6, -1)
    pxf = px.astype(jnp.float32)
    pyf = py.astype(jnp.float32)

    inb = (px > 1) & (px < W - 1) & (py > 1) & (py < H - 1)
    pxc = jnp.clip(px, 1, W - 2)
    pyc = jnp.clip(py, 1, H - 2)

    rowv = pyc + jnp.where(lanes == 2, 1, jnp.where(lanes == 3, -1, 0))
    colv = pxc + jnp.where(lanes == 0, 1, jnp.where(lanes == 1, -1, 0))
    v4 = plsc.load_gather(hm_v, [rowv, colv])
    zero = jnp.zeros((L,), jnp.float32)
    dx = jnp.sum(jnp.where(lanes == 0, v4, jnp.where(lanes == 1, -v4, zero)))
    dy = jnp.sum(jnp.where(lanes == 2, v4, jnp.where(lanes == 3, -v4, zero)))

    rx = pxf + jnp.where(inb, jnp.sign(dx) * 0.25, 0.0)
    ry = pyf + jnp.where(inb, jnp.sign(dy) * 0.25, 0.0)

    cp_c.wait()
    cp_s.wait()
    zer16 = jnp.zeros((L,), jnp.int32)
    pair = jnp.minimum(lanes, 1)
    cv = plsc.load_gather(cen_v, [zer16, pair])
    sv = plsc.load_gather(scl_v, [zer16, pair])
    cx = cv[0]
    cy = cv[1]
    scx = sv[0] * 200.0
    scy = sv[1] * 200.0
    tx = rx * (scx * (1.0 / W)) + cx - scx * 0.5
    ty = ry * (scy * (1.0 / H)) + cy - scy * 0.5

    res_v[...] = jnp.where(
        lanes == 0, tx,
        jnp.where(lanes == 1, ty, jnp.where(lanes == 2, m, zero)))
    pltpu.sync_copy(res_v, shared.at[pl.ds(s * L, L)])
    plsc.subcore_barrier()

    @pl.when(s == 0)
    def _():
        pltpu.sync_copy(shared, big_v)
        half = (lanes >> 1) * L + (lanes & 1)
        ab = plsc.load_gather(big_v, [half])
        cd = plsc.load_gather(big_v, [half + 8 * L])
        stage_v[pl.ds(0, L)] = ab
        stage_v[pl.ds(L, L)] = cd
        pltpu.sync_copy(stage_v, out_hbm)

    @pl.when(s == 1)
    def _():
        pltpu.sync_copy(shared, big_v)
        mv = plsc.load_gather(big_v, [lanes * L + 2])
        res_v[...] = mv
        pltpu.sync_copy(res_v, mv_hbm)


@jax.jit
def kernel(heatmaps, center, scale):
    mesh = plsc.VectorSubcoreMesh(
        core_axis_name="c", subcore_axis_name="s", num_cores=1)
    run = pl.kernel(
        _decode_kernel,
        out_type=(jax.ShapeDtypeStruct((2 * L,), jnp.float32),
                  jax.ShapeDtypeStruct((L,), jnp.float32)),
        mesh=mesh,
        scratch_types=[
            pltpu.VMEM((H, W), jnp.float32),
            pltpu.VMEM((1, 2), jnp.float32),
            pltpu.VMEM((1, 2), jnp.float32),
            pltpu.VMEM((L,), jnp.float32),
            pltpu.VMEM((K * L,), jnp.float32),
            pltpu.VMEM((2 * L,), jnp.float32),
            pltpu.VMEM_SHARED((K * L,), jnp.float32),
            pltpu.SemaphoreType.DMA,
            pltpu.SemaphoreType.DMA,
            pltpu.SemaphoreType.DMA,
        ],
        compiler_params=pltpu.CompilerParams(needs_layout_passes=False),
    )
    out_flat, mv_flat = run(heatmaps, center, scale)
    return out_flat.reshape(N, K, 2), mv_flat.reshape(N, K, 1)
